# baseline (speedup 1.0000x reference)
import functools

import jax
import jax.numpy as jnp
from jax import lax
from jax.experimental import pallas as pl
from jax.experimental.pallas import tpu as pltpu
from jax.experimental.pallas import tpu_sc as plsc

NC, NS, LANES = 2, 16, 16
NW = NC * NS
C = 5
LROW = 4194304
NTOT = C * LROW
PER_TILE = LROW // NW
CHUNK = 8192
NCHUNK = PER_TILE // CHUNK
RANK = 4152360
TSTRIDE = 257
LSTRIDE = C * TSTRIDE

_MESH = plsc.VectorSubcoreMesh(
    core_axis_name="c", subcore_axis_name="s", num_cores=NC, num_subcores=NS)
_PARAMS = pltpu.CompilerParams(needs_layout_passes=False)


def _stream_rows(x_hbm, buf, sems, wid, body, row_prologue, row_epilogue):
  for r in range(C):
    row_prologue(r)
    base = r * LROW + wid * PER_TILE
    pltpu.async_copy(x_hbm.at[pl.ds(base, CHUNK)], buf.at[0], sems[0])
    pltpu.async_copy(x_hbm.at[pl.ds(base + CHUNK, CHUNK)], buf.at[1], sems[1])

    @pl.loop(0, NCHUNK, step=2)
    def _(g2):
      for b in range(2):
        g = g2 + b
        pltpu.make_async_copy(
            x_hbm.at[pl.ds(0, CHUNK)], buf.at[b], sems[b]).wait()

        @pl.loop(0, CHUNK // LANES, unroll=8)
        def _(i):
          body(r, b, i)

        nxt = g + 2

        @pl.when(nxt < NCHUNK)
        def _():
          pltpu.async_copy(
              x_hbm.at[pl.ds(base + nxt * CHUNK, CHUNK)], buf.at[b], sems[b])

    row_epilogue(r)


def _make_refine(shift, nbits, masked):
  nbins = 1 << nbits
  stride = nbins + 1
  hsz = LANES * stride + LANES

  scratch = [
      pltpu.VMEM((2, CHUNK), jnp.float32),
      pltpu.VMEM((hsz,), jnp.int32),
      pltpu.VMEM((nbins,), jnp.int32),
  ]
  if masked:
    scratch.append(pltpu.VMEM((C * LANES,), jnp.int32))
  scratch += [pltpu.SemaphoreType.DMA, pltpu.SemaphoreType.DMA]

  @functools.partial(
      pl.kernel,
      out_type=jax.ShapeDtypeStruct((NW * C * nbins,), jnp.int32),
      mesh=_MESH,
      compiler_params=_PARAMS,
      scratch_types=scratch,
  )
  def kfn(*args):
    if masked:
      x_hbm, prevp_hbm, out_hbm, buf, hist, merged, prevp_v, sem0, sem1 = args
    else:
      x_hbm, out_hbm, buf, hist, merged, sem0, sem1 = args
      prevp_v = None
    wid = lax.axis_index("c") * NS + lax.axis_index("s")
    if masked:
      pltpu.sync_copy(prevp_hbm, prevp_v)
    lane_base = lax.iota(jnp.int32, LANES) * stride
    ones = jnp.ones((LANES,), jnp.int32)
    zer = jnp.zeros((LANES,), jnp.int32)
    row_state = {}

    def pro(r):
      @pl.loop(0, hsz // LANES, unroll=8)
      def _(j):
        hist[pl.ds(j * LANES, LANES)] = zer
      if masked:
        row_state["prev"] = prevp_v[pl.ds(r * LANES, LANES)]

    def body(r, b, i):
      v = buf[b, pl.ds(i * LANES, LANES)]
      p = lax.bitcast_convert_type(v, jnp.int32) & 0x7FFFFFFF
      digit = ((p >> shift) & (nbins - 1)) + lane_base
      if masked:
        m = (p >> (shift + nbits)) == row_state["prev"]
        plsc.addupdate_scatter(hist, [digit], ones, mask=m)
      else:
        plsc.addupdate_scatter(hist, [digit], ones)

    def epi(r):
      @pl.loop(0, nbins // LANES)
      def _(jb):
        o = jb * LANES
        acc = hist[pl.ds(o, LANES)]
        for l in range(1, LANES):
          acc = acc + hist[pl.ds(l * stride + o, LANES)]
        merged[pl.ds(o, LANES)] = acc
      pltpu.sync_copy(merged, out_hbm.at[pl.ds((wid * C + r) * nbins, nbins)])

    _stream_rows(x_hbm, buf, (sem0, sem1), wid, body, pro, epi)

  return kfn


def _replicate_table(src_v, rep_v, cast_to=None):
  for r in range(C):
    for jb in range(256 // LANES):
      tv = src_v[pl.ds(r * 256 + jb * LANES, LANES)]
      if cast_to is not None:
        tv = lax.bitcast_convert_type(tv, cast_to)
      for l in range(LANES):
        rep_v[pl.ds(l * LSTRIDE + r * TSTRIDE + jb * LANES, LANES)] = tv


def _exact_bin(v, rv, lane_base_r, trep):
  a = jnp.abs(v)
  t = a * rv
  cmin = jnp.minimum(t, jnp.float32(1.0))
  d = cmin * jnp.float32(255.0)
  kp = d.astype(jnp.int32) + lane_base_r
  tb = plsc.load_gather(trep, [kp])
  pa = lax.bitcast_convert_type(a, jnp.int32)
  return kp - (pa < tb).astype(jnp.int32)


def _make_hist256():
  hsz = LANES * LSTRIDE + LANES

  @functools.partial(
      pl.kernel,
      out_type=jax.ShapeDtypeStruct((NW * C * 256,), jnp.int32),
      mesh=_MESH,
      compiler_params=_PARAMS,
      scratch_types=[
          pltpu.VMEM((2, CHUNK), jnp.float32),
          pltpu.VMEM((hsz,), jnp.int32),
          pltpu.VMEM((C * 256,), jnp.int32),
          pltpu.VMEM((LANES * LSTRIDE,), jnp.int32),
          pltpu.VMEM((C * LANES,), jnp.float32),
          pltpu.VMEM((256,), jnp.int32),
          pltpu.SemaphoreType.DMA,
          pltpu.SemaphoreType.DMA,
      ],
  )
  def kfn(x_hbm, tbits_hbm, rbias_hbm, out_hbm, buf, hist, tstage, trep,
          rbias_v, merged, sem0, sem1):
    wid = lax.axis_index("c") * NS + lax.axis_index("s")
    pltpu.sync_copy(tbits_hbm, tstage)
    pltpu.sync_copy(rbias_hbm, rbias_v)
    _replicate_table(tstage, trep)
    lane_base = lax.iota(jnp.int32, LANES) * LSTRIDE
    ones = jnp.ones((LANES,), jnp.int32)
    zer = jnp.zeros((LANES,), jnp.int32)

    @pl.loop(0, hsz // LANES, unroll=8)
    def _(j):
      hist[pl.ds(j * LANES, LANES)] = zer

    row_state = {}

    def pro(r):
      row_state["rv"] = rbias_v[pl.ds(r * LANES, LANES)]
      row_state["base"] = lane_base + (r * TSTRIDE)

    def body(r, b, i):
      v = buf[b, pl.ds(i * LANES, LANES)]
      k = _exact_bin(v, row_state["rv"], row_state["base"], trep)
      plsc.addupdate_scatter(hist, [k], ones)

    def epi(r):
      @pl.loop(0, 256 // LANES)
      def _(jb):
        o = r * TSTRIDE + jb * LANES
        acc = hist[pl.ds(o, LANES)]
        for l in range(1, LANES):
          acc = acc + hist[pl.ds(l * LSTRIDE + o, LANES)]
        merged[pl.ds(jb * LANES, LANES)] = acc
      pltpu.sync_copy(merged, out_hbm.at[pl.ds((wid * C + r) * 256, 256)])

    _stream_rows(x_hbm, buf, (sem0, sem1), wid, body, pro, epi)

  return kfn


def _make_apply():
  @functools.partial(
      pl.kernel,
      out_type=jax.ShapeDtypeStruct((NTOT,), jnp.float32),
      mesh=_MESH,
      compiler_params=_PARAMS,
      scratch_types=[
          pltpu.VMEM((2, CHUNK), jnp.float32),
          pltpu.VMEM((2, CHUNK), jnp.float32),
          pltpu.VMEM((C * 256,), jnp.int32),
          pltpu.VMEM((LANES * LSTRIDE,), jnp.int32),
          pltpu.VMEM((C * 256,), jnp.float32),
          pltpu.VMEM((LANES * LSTRIDE,), jnp.float32),
          pltpu.VMEM((C * LANES,), jnp.float32),
          pltpu.SemaphoreType.DMA,
          pltpu.SemaphoreType.DMA,
          pltpu.SemaphoreType.DMA,
          pltpu.SemaphoreType.DMA,
      ],
  )
  def kfn(x_hbm, tbits_hbm, mtab_hbm, rbias_hbm, out_hbm, buf, obuf, tstage,
          trep, mstage, mrep, rbias_v, si0, si1, so0, so1):
    wid = lax.axis_index("c") * NS + lax.axis_index("s")
    pltpu.sync_copy(tbits_hbm, tstage)
    pltpu.sync_copy(mtab_hbm, mstage)
    pltpu.sync_copy(rbias_hbm, rbias_v)
    _replicate_table(tstage, trep)
    _replicate_table(mstage, mrep)
    lane_base = lax.iota(jnp.int32, LANES) * LSTRIDE
    isems = (si0, si1)
    osems = (so0, so1)

    for r in range(C):
      rv = rbias_v[pl.ds(r * LANES, LANES)]
      lane_base_r = lane_base + (r * TSTRIDE)
      base = r * LROW + wid * PER_TILE
      pltpu.async_copy(x_hbm.at[pl.ds(base, CHUNK)], buf.at[0], isems[0])
      pltpu.async_copy(x_hbm.at[pl.ds(base + CHUNK, CHUNK)], buf.at[1],
                       isems[1])

      @pl.loop(0, NCHUNK, step=2)
      def _(g2):
        for b in range(2):
          g = g2 + b
          pltpu.make_async_copy(
              x_hbm.at[pl.ds(0, CHUNK)], buf.at[b], isems[b]).wait()

          @pl.when(g >= 2)
          def _():
            pltpu.make_async_copy(
                obuf.at[b], out_hbm.at[pl.ds(0, CHUNK)], osems[b]).wait()

          @pl.loop(0, CHUNK // LANES, unroll=8)
          def _(i):
            v = buf[b, pl.ds(i * LANES, LANES)]
            k = _exact_bin(v, rv, lane_base_r, trep)
            gt = plsc.load_gather(mrep, [k])
            obuf[b, pl.ds(i * LANES, LANES)] = v * gt

          pltpu.async_copy(
              obuf.at[b], out_hbm.at[pl.ds(base + g * CHUNK, CHUNK)], osems[b])
          nxt = g + 2

          @pl.when(nxt < NCHUNK)
          def _():
            pltpu.async_copy(
                x_hbm.at[pl.ds(base + nxt * CHUNK, CHUNK)], buf.at[b],
                isems[b])

      for b in range(2):
        pltpu.make_async_copy(
            obuf.at[b], out_hbm.at[pl.ds(0, CHUNK)], osems[b]).wait()

  return kfn


_L1 = _make_refine(20, 11, masked=False)
_L2 = _make_refine(9, 11, masked=True)
_L3 = _make_refine(0, 9, masked=True)
_H256 = _make_hist256()
_APPLY = _make_apply()


def _pick(cnt, rank):
  cum = jnp.cumsum(cnt, axis=1)
  b = jnp.argmax(cum >= (rank[:, None] + 1), axis=1).astype(jnp.int32)
  cumprev = jnp.take_along_axis(cum - cnt, b[:, None], axis=1)[:, 0]
  return b, rank - cumprev


def _ref_bin_of_bits(pb, maxv):
  v = lax.bitcast_convert_type(pb, jnp.float32)
  t = v / maxv
  n = t * jnp.float32(8.0)
  cc = jnp.clip(n, jnp.float32(0.0), jnp.float32(8.0))
  ix = ((cc / jnp.float32(8.0)) * jnp.float32(255.0)).astype(jnp.int32)
  return jnp.clip(ix, 0, 255)


def _bin_thresholds(maxv):
  ks = jnp.arange(256, dtype=jnp.int32)[None, :]
  lo = jnp.zeros((C, 256), jnp.int32)
  hi = jnp.full((C, 256), 0x7F800000, jnp.int32)
  mv = maxv[:, None]
  for _ in range(31):
    mid = lo + ((hi - lo) >> 1)
    ok = _ref_bin_of_bits(mid, mv) >= ks
    hi = jnp.where(ok, mid, hi)
    lo = jnp.where(ok, lo, mid + 1)
  return hi.at[:, 0].set(0)


def kernel(x, hist, logp_ref):
  xf = jnp.reshape(x, (-1,))

  cnt1 = jnp.sum(jnp.reshape(_L1(xf), (NW, C, 2048)), axis=0)
  b1, r1 = _pick(cnt1, jnp.full((C,), RANK, jnp.int32))

  prev2 = jnp.reshape(jnp.broadcast_to(b1[:, None], (C, LANES)), (-1,))
  cnt2 = jnp.sum(jnp.reshape(_L2(xf, prev2), (NW, C, 2048)), axis=0)
  b2, r2 = _pick(cnt2, r1)

  prev3 = jnp.reshape(
      jnp.broadcast_to(((b1 << 11) | b2)[:, None], (C, LANES)), (-1,))
  cnt3 = jnp.sum(jnp.reshape(_L3(xf, prev3), (NW, C, 512)), axis=0)
  b3, _ = _pick(cnt3, r2)

  bits = (b1 << 20) | (b2 << 9) | b3
  maxv = jnp.maximum(
      lax.bitcast_convert_type(bits.astype(jnp.int32), jnp.float32),
      jnp.float32(1e-8))

  tbits = jnp.reshape(_bin_thresholds(maxv), (-1,))
  rbias = (jnp.float32(1.0) / maxv) * jnp.float32(1.0 + 2.0**-20)
  rbias_b = jnp.reshape(jnp.broadcast_to(rbias[:, None], (C, LANES)), (-1,))

  counts = jnp.sum(
      jnp.reshape(_H256(xf, tbits, rbias_b), (NW, C, 256)),
      axis=0).astype(jnp.float32)

  hist2 = (1.0 - 0.02) * hist + 0.02 * counts
  sm = hist2 + 1e-8
  logp_obs = jnp.log(sm / jnp.sum(sm, axis=-1, keepdims=True))
  mask_tab = jax.nn.sigmoid(-1.0 * ((logp_ref - logp_obs) - (-2.0)))

  out = _APPLY(xf, tbits, jnp.reshape(mask_tab, (-1,)), rbias_b)
  return jnp.reshape(out, x.shape)


# speedup vs baseline: 2.7263x; 2.7263x over previous
import functools

import jax
import jax.numpy as jnp
from jax import lax
from jax.experimental import pallas as pl
from jax.experimental.pallas import tpu as pltpu
from jax.experimental.pallas import tpu_sc as plsc

NC, NS, LANES = 2, 16, 16
NW = NC * NS
C = 5
LROW = 4194304
NTOT = C * LROW
PER_TILE = LROW // NW
CHUNK = 8192
NCHUNK = PER_TILE // CHUNK
RANK = 4152360
TSTRIDE = 257
LSTRIDE = C * TSTRIDE

_MESH = plsc.VectorSubcoreMesh(
    core_axis_name="c", subcore_axis_name="s", num_cores=NC, num_subcores=NS)
_PARAMS = pltpu.CompilerParams(needs_layout_passes=False)


def _stream_rows(x_hbm, buf, sems, wid, body2, row_prologue, row_epilogue):
  for r in range(C):
    row_prologue(r)
    base = r * LROW + wid * PER_TILE
    pltpu.async_copy(x_hbm.at[pl.ds(base, CHUNK)], buf.at[0], sems[0])
    pltpu.async_copy(x_hbm.at[pl.ds(base + CHUNK, CHUNK)], buf.at[1], sems[1])

    @pl.loop(0, NCHUNK, step=2)
    def _(g2):
      for b in range(2):
        g = g2 + b
        pltpu.make_async_copy(
            x_hbm.at[pl.ds(0, CHUNK)], buf.at[b], sems[b]).wait()

        @plsc.parallel_loop(0, CHUNK // LANES, step=2, unroll=4)
        def _(i):
          body2(r, b, i)

        nxt = g + 2

        @pl.when(nxt < NCHUNK)
        def _():
          pltpu.async_copy(
              x_hbm.at[pl.ds(base + nxt * CHUNK, CHUNK)], buf.at[b], sems[b])

    row_epilogue(r)


def _make_refine(shift, nbits, masked):
  nbins = 1 << nbits
  stride = nbins + 1
  hsz = LANES * stride + LANES

  scratch = [
      pltpu.VMEM((2, CHUNK), jnp.float32),
      pltpu.VMEM((hsz,), jnp.int32),
      pltpu.VMEM((hsz,), jnp.int32),
      pltpu.VMEM((nbins,), jnp.int32),
  ]
  if masked:
    scratch.append(pltpu.VMEM((C * LANES,), jnp.int32))
  scratch += [pltpu.SemaphoreType.DMA, pltpu.SemaphoreType.DMA]

  @functools.partial(
      pl.kernel,
      out_type=jax.ShapeDtypeStruct((NW * C * nbins,), jnp.int32),
      mesh=_MESH,
      compiler_params=_PARAMS,
      scratch_types=scratch,
  )
  def kfn(*args):
    if masked:
      (x_hbm, prevp_hbm, out_hbm, buf, hist0, hist1, merged, prevp_v,
       sem0, sem1) = args
    else:
      x_hbm, out_hbm, buf, hist0, hist1, merged, sem0, sem1 = args
      prevp_v = None
    wid = lax.axis_index("c") * NS + lax.axis_index("s")
    if masked:
      pltpu.sync_copy(prevp_hbm, prevp_v)
    lane_base = lax.iota(jnp.int32, LANES) * stride
    ones = jnp.ones((LANES,), jnp.int32)
    zer = jnp.zeros((LANES,), jnp.int32)
    row_state = {}

    def pro(r):
      @plsc.parallel_loop(0, hsz // LANES, unroll=8)
      def _(j):
        hist0[pl.ds(j * LANES, LANES)] = zer
        hist1[pl.ds(j * LANES, LANES)] = zer
      if masked:
        row_state["prev"] = prevp_v[pl.ds(r * LANES, LANES)]

    def one(hist, b, i):
      v = buf[b, pl.ds(i * LANES, LANES)]
      p = lax.bitcast_convert_type(v, jnp.int32) & 0x7FFFFFFF
      digit = ((p >> shift) & (nbins - 1)) + lane_base
      if masked:
        m = (p >> (shift + nbits)) == row_state["prev"]
        plsc.addupdate_scatter(hist, [digit], ones, mask=m)
      else:
        plsc.addupdate_scatter(hist, [digit], ones)

    def body2(r, b, i):
      one(hist0, b, i)
      one(hist1, b, i + 1)

    def epi(r):
      @plsc.parallel_loop(0, nbins // LANES, unroll=2)
      def _(jb):
        o = jb * LANES
        acc = hist0[pl.ds(o, LANES)] + hist1[pl.ds(o, LANES)]
        for l in range(1, LANES):
          acc = acc + hist0[pl.ds(l * stride + o, LANES)]
          acc = acc + hist1[pl.ds(l * stride + o, LANES)]
        merged[pl.ds(o, LANES)] = acc
      pltpu.sync_copy(merged, out_hbm.at[pl.ds((wid * C + r) * nbins, nbins)])

    _stream_rows(x_hbm, buf, (sem0, sem1), wid, body2, pro, epi)

  return kfn


def _replicate_table(src_v, rep_v):
  for r in range(C):
    for jb in range(256 // LANES):
      tv = src_v[pl.ds(r * 256 + jb * LANES, LANES)]
      for l in range(LANES):
        rep_v[pl.ds(l * LSTRIDE + r * TSTRIDE + jb * LANES, LANES)] = tv


def _exact_bin(v, rv, lane_base_r, trep):
  a = jnp.abs(v)
  t = a * rv
  cmin = jnp.minimum(t, jnp.float32(1.0))
  d = cmin * jnp.float32(255.0)
  kp = d.astype(jnp.int32) + lane_base_r
  tb = plsc.load_gather(trep, [kp])
  pa = lax.bitcast_convert_type(a, jnp.int32)
  return kp - (pa < tb).astype(jnp.int32)


def _make_hist256():
  hsz = LANES * LSTRIDE + LANES

  @functools.partial(
      pl.kernel,
      out_type=jax.ShapeDtypeStruct((NW * C * 256,), jnp.int32),
      mesh=_MESH,
      compiler_params=_PARAMS,
      scratch_types=[
          pltpu.VMEM((2, CHUNK), jnp.float32),
          pltpu.VMEM((hsz,), jnp.int32),
          pltpu.VMEM((hsz,), jnp.int32),
          pltpu.VMEM((C * 256,), jnp.int32),
          pltpu.VMEM((LANES * LSTRIDE,), jnp.int32),
          pltpu.VMEM((C * LANES,), jnp.float32),
          pltpu.VMEM((256,), jnp.int32),
          pltpu.SemaphoreType.DMA,
          pltpu.SemaphoreType.DMA,
      ],
  )
  def kfn(x_hbm, tbits_hbm, rbias_hbm, out_hbm, buf, hist0, hist1, tstage,
          trep, rbias_v, merged, sem0, sem1):
    wid = lax.axis_index("c") * NS + lax.axis_index("s")
    pltpu.sync_copy(tbits_hbm, tstage)
    pltpu.sync_copy(rbias_hbm, rbias_v)
    _replicate_table(tstage, trep)
    lane_base = lax.iota(jnp.int32, LANES) * LSTRIDE
    ones = jnp.ones((LANES,), jnp.int32)
    zer = jnp.zeros((LANES,), jnp.int32)

    @plsc.parallel_loop(0, hsz // LANES, unroll=8)
    def _(j):
      hist0[pl.ds(j * LANES, LANES)] = zer
      hist1[pl.ds(j * LANES, LANES)] = zer

    row_state = {}

    def pro(r):
      row_state["rv"] = rbias_v[pl.ds(r * LANES, LANES)]
      row_state["base"] = lane_base + (r * TSTRIDE)

    def one(hist, b, i):
      v = buf[b, pl.ds(i * LANES, LANES)]
      k = _exact_bin(v, row_state["rv"], row_state["base"], trep)
      plsc.addupdate_scatter(hist, [k], ones)

    def body2(r, b, i):
      one(hist0, b, i)
      one(hist1, b, i + 1)

    def epi(r):
      @plsc.parallel_loop(0, 256 // LANES)
      def _(jb):
        o = r * TSTRIDE + jb * LANES
        acc = hist0[pl.ds(o, LANES)] + hist1[pl.ds(o, LANES)]
        for l in range(1, LANES):
          acc = acc + hist0[pl.ds(l * LSTRIDE + o, LANES)]
          acc = acc + hist1[pl.ds(l * LSTRIDE + o, LANES)]
        merged[pl.ds(jb * LANES, LANES)] = acc
      pltpu.sync_copy(merged, out_hbm.at[pl.ds((wid * C + r) * 256, 256)])

    _stream_rows(x_hbm, buf, (sem0, sem1), wid, body2, pro, epi)

  return kfn


def _make_apply():
  @functools.partial(
      pl.kernel,
      out_type=jax.ShapeDtypeStruct((NTOT,), jnp.float32),
      mesh=_MESH,
      compiler_params=_PARAMS,
      scratch_types=[
          pltpu.VMEM((2, CHUNK), jnp.float32),
          pltpu.VMEM((2, CHUNK), jnp.float32),
          pltpu.VMEM((C * 256,), jnp.int32),
          pltpu.VMEM((LANES * LSTRIDE,), jnp.int32),
          pltpu.VMEM((C * 256,), jnp.float32),
          pltpu.VMEM((LANES * LSTRIDE,), jnp.float32),
          pltpu.VMEM((C * LANES,), jnp.float32),
          pltpu.SemaphoreType.DMA,
          pltpu.SemaphoreType.DMA,
          pltpu.SemaphoreType.DMA,
          pltpu.SemaphoreType.DMA,
      ],
  )
  def kfn(x_hbm, tbits_hbm, mtab_hbm, rbias_hbm, out_hbm, buf, obuf, tstage,
          trep, mstage, mrep, rbias_v, si0, si1, so0, so1):
    wid = lax.axis_index("c") * NS + lax.axis_index("s")
    pltpu.sync_copy(tbits_hbm, tstage)
    pltpu.sync_copy(mtab_hbm, mstage)
    pltpu.sync_copy(rbias_hbm, rbias_v)
    _replicate_table(tstage, trep)
    _replicate_table(mstage, mrep)
    lane_base = lax.iota(jnp.int32, LANES) * LSTRIDE
    isems = (si0, si1)
    osems = (so0, so1)

    for r in range(C):
      rv = rbias_v[pl.ds(r * LANES, LANES)]
      lane_base_r = lane_base + (r * TSTRIDE)
      base = r * LROW + wid * PER_TILE
      pltpu.async_copy(x_hbm.at[pl.ds(base, CHUNK)], buf.at[0], isems[0])
      pltpu.async_copy(x_hbm.at[pl.ds(base + CHUNK, CHUNK)], buf.at[1],
                       isems[1])

      @pl.loop(0, NCHUNK, step=2)
      def _(g2):
        for b in range(2):
          g = g2 + b
          pltpu.make_async_copy(
              x_hbm.at[pl.ds(0, CHUNK)], buf.at[b], isems[b]).wait()

          @pl.when(g >= 2)
          def _():
            pltpu.make_async_copy(
                obuf.at[b], out_hbm.at[pl.ds(0, CHUNK)], osems[b]).wait()

          @plsc.parallel_loop(0, CHUNK // LANES, unroll=8)
          def _(i):
            v = buf[b, pl.ds(i * LANES, LANES)]
            k = _exact_bin(v, rv, lane_base_r, trep)
            gt = plsc.load_gather(mrep, [k])
            obuf[b, pl.ds(i * LANES, LANES)] = v * gt

          pltpu.async_copy(
              obuf.at[b], out_hbm.at[pl.ds(base + g * CHUNK, CHUNK)], osems[b])
          nxt = g + 2

          @pl.when(nxt < NCHUNK)
          def _():
            pltpu.async_copy(
                x_hbm.at[pl.ds(base + nxt * CHUNK, CHUNK)], buf.at[b],
                isems[b])

      for b in range(2):
        pltpu.make_async_copy(
            obuf.at[b], out_hbm.at[pl.ds(0, CHUNK)], osems[b]).wait()

  return kfn


_L1 = _make_refine(20, 11, masked=False)
_L2 = _make_refine(9, 11, masked=True)
_L3 = _make_refine(0, 9, masked=True)
_H256 = _make_hist256()
_APPLY = _make_apply()


def _pick(cnt, rank):
  cum = jnp.cumsum(cnt, axis=1)
  b = jnp.argmax(cum >= (rank[:, None] + 1), axis=1).astype(jnp.int32)
  cumprev = jnp.take_along_axis(cum - cnt, b[:, None], axis=1)[:, 0]
  return b, rank - cumprev


def _ref_bin_of_bits(pb, maxv):
  v = lax.bitcast_convert_type(pb, jnp.float32)
  t = v / maxv
  n = t * jnp.float32(8.0)
  cc = jnp.clip(n, jnp.float32(0.0), jnp.float32(8.0))
  ix = ((cc / jnp.float32(8.0)) * jnp.float32(255.0)).astype(jnp.int32)
  return jnp.clip(ix, 0, 255)


def _bin_thresholds(maxv):
  ks = jnp.arange(256, dtype=jnp.int32)[None, :]
  lo = jnp.zeros((C, 256), jnp.int32)
  hi = jnp.full((C, 256), 0x7F800000, jnp.int32)
  mv = maxv[:, None]
  for _ in range(31):
    mid = lo + ((hi - lo) >> 1)
    ok = _ref_bin_of_bits(mid, mv) >= ks
    hi = jnp.where(ok, mid, hi)
    lo = jnp.where(ok, lo, mid + 1)
  return hi.at[:, 0].set(0)


def kernel(x, hist, logp_ref):
  xf = jnp.reshape(x, (-1,))

  cnt1 = jnp.sum(jnp.reshape(_L1(xf), (NW, C, 2048)), axis=0)
  b1, r1 = _pick(cnt1, jnp.full((C,), RANK, jnp.int32))

  prev2 = jnp.reshape(jnp.broadcast_to(b1[:, None], (C, LANES)), (-1,))
  cnt2 = jnp.sum(jnp.reshape(_L2(xf, prev2), (NW, C, 2048)), axis=0)
  b2, r2 = _pick(cnt2, r1)

  prev3 = jnp.reshape(
      jnp.broadcast_to(((b1 << 11) | b2)[:, None], (C, LANES)), (-1,))
  cnt3 = jnp.sum(jnp.reshape(_L3(xf, prev3), (NW, C, 512)), axis=0)
  b3, _ = _pick(cnt3, r2)

  bits = (b1 << 20) | (b2 << 9) | b3
  maxv = jnp.maximum(
      lax.bitcast_convert_type(bits.astype(jnp.int32), jnp.float32),
      jnp.float32(1e-8))

  tbits = jnp.reshape(_bin_thresholds(maxv), (-1,))
  rbias = (jnp.float32(1.0) / maxv) * jnp.float32(1.0 + 2.0**-20)
  rbias_b = jnp.reshape(jnp.broadcast_to(rbias[:, None], (C, LANES)), (-1,))

  counts = jnp.sum(
      jnp.reshape(_H256(xf, tbits, rbias_b), (NW, C, 256)),
      axis=0).astype(jnp.float32)

  hist2 = (1.0 - 0.02) * hist + 0.02 * counts
  sm = hist2 + 1e-8
  logp_obs = jnp.log(sm / jnp.sum(sm, axis=-1, keepdims=True))
  mask_tab = jax.nn.sigmoid(-1.0 * ((logp_ref - logp_obs) - (-2.0)))

  out = _APPLY(xf, tbits, jnp.reshape(mask_tab, (-1,)), rbias_b)
  return jnp.reshape(out, x.shape)


# speedup vs baseline: 5.0832x; 1.8645x over previous
import functools

import jax
import jax.numpy as jnp
from jax import lax
from jax.experimental import pallas as pl
from jax.experimental.pallas import tpu as pltpu
from jax.experimental.pallas import tpu_sc as plsc

NC, NS, LANES = 2, 16, 16
NW = NC * NS
C = 5
LROW = 4194304
NTOT = C * LROW
PER_TILE = LROW // NW
CHUNK = 8192
NCHUNK = PER_TILE // CHUNK
RANK = 4152360
TSTRIDE = 257
LSTRIDE = C * TSTRIDE

_MESH = plsc.VectorSubcoreMesh(
    core_axis_name="c", subcore_axis_name="s", num_cores=NC, num_subcores=NS)
_PARAMS = pltpu.CompilerParams(needs_layout_passes=False)


def _chunk_slice(x_hbm, wid, r, g):
  q = (r * 8) + (g >> 1)
  b = (q * 13) >> 6
  c = q - b * 5
  off = wid * (2 * CHUNK) + (g & 1) * CHUNK
  return x_hbm.at[b, pl.ds(c, 1), pl.ds(off, CHUNK)]


def _stream_rows(x_hbm, buf, sems, wid, body2, row_prologue, row_epilogue):
  for r in range(C):
    row_prologue(r)
    pltpu.async_copy(_chunk_slice(x_hbm, wid, r, 0), buf.at[0], sems[0])
    pltpu.async_copy(_chunk_slice(x_hbm, wid, r, 1), buf.at[1], sems[1])

    @pl.loop(0, NCHUNK, step=2)
    def _(g2):
      for b in range(2):
        g = g2 + b
        pltpu.make_async_copy(
            x_hbm.at[0, pl.ds(0, 1), pl.ds(0, CHUNK)], buf.at[b], sems[b]).wait()

        @plsc.parallel_loop(0, CHUNK // LANES, step=2, unroll=4)
        def _(i):
          body2(r, b, i)

        nxt = g + 2

        @pl.when(nxt < NCHUNK)
        def _():
          pltpu.async_copy(
              _chunk_slice(x_hbm, wid, r, nxt), buf.at[b], sems[b])

    row_epilogue(r)


def _make_refine(shift, nbits, masked):
  nbins = 1 << nbits
  stride = nbins + 1
  hsz = LANES * stride + LANES

  scratch = [
      pltpu.VMEM((2, 1, CHUNK), jnp.float32),
      pltpu.VMEM((hsz,), jnp.int32),
      pltpu.VMEM((hsz,), jnp.int32),
      pltpu.VMEM((nbins,), jnp.int32),
  ]
  if masked:
    scratch.append(pltpu.VMEM((C * LANES,), jnp.int32))
  scratch += [pltpu.SemaphoreType.DMA, pltpu.SemaphoreType.DMA]

  @functools.partial(
      pl.kernel,
      out_type=jax.ShapeDtypeStruct((NW * C * nbins,), jnp.int32),
      mesh=_MESH,
      compiler_params=_PARAMS,
      scratch_types=scratch,
  )
  def kfn(*args):
    if masked:
      (x_hbm, prevp_hbm, out_hbm, buf, hist0, hist1, merged, prevp_v,
       sem0, sem1) = args
    else:
      x_hbm, out_hbm, buf, hist0, hist1, merged, sem0, sem1 = args
      prevp_v = None
    wid = lax.axis_index("c") * NS + lax.axis_index("s")
    if masked:
      pltpu.sync_copy(prevp_hbm, prevp_v)
    lane_base = lax.iota(jnp.int32, LANES) * stride
    ones = jnp.ones((LANES,), jnp.int32)
    zer = jnp.zeros((LANES,), jnp.int32)
    row_state = {}

    def pro(r):
      @plsc.parallel_loop(0, hsz // LANES, unroll=8)
      def _(j):
        hist0[pl.ds(j * LANES, LANES)] = zer
        hist1[pl.ds(j * LANES, LANES)] = zer
      if masked:
        row_state["prev"] = prevp_v[pl.ds(r * LANES, LANES)]

    def one(hist, b, i):
      v = buf[b, 0, pl.ds(i * LANES, LANES)]
      p = lax.bitcast_convert_type(v, jnp.int32) & 0x7FFFFFFF
      digit = ((p >> shift) & (nbins - 1)) + lane_base
      if masked:
        m = (p >> (shift + nbits)) == row_state["prev"]
        plsc.addupdate_scatter(hist, [digit], ones, mask=m)
      else:
        plsc.addupdate_scatter(hist, [digit], ones)

    def body2(r, b, i):
      one(hist0, b, i)
      one(hist1, b, i + 1)

    def epi(r):
      @plsc.parallel_loop(0, nbins // LANES, unroll=2)
      def _(jb):
        o = jb * LANES
        acc = hist0[pl.ds(o, LANES)] + hist1[pl.ds(o, LANES)]
        for l in range(1, LANES):
          acc = acc + hist0[pl.ds(l * stride + o, LANES)]
          acc = acc + hist1[pl.ds(l * stride + o, LANES)]
        merged[pl.ds(o, LANES)] = acc
      pltpu.sync_copy(merged, out_hbm.at[pl.ds((wid * C + r) * nbins, nbins)])

    _stream_rows(x_hbm, buf, (sem0, sem1), wid, body2, pro, epi)

  return kfn


def _replicate_table(src_v, rep_v):
  for r in range(C):
    for jb in range(256 // LANES):
      tv = src_v[pl.ds(r * 256 + jb * LANES, LANES)]
      for l in range(LANES):
        rep_v[pl.ds(l * LSTRIDE + r * TSTRIDE + jb * LANES, LANES)] = tv


def _exact_bin(v, rv, lane_base_r, trep):
  a = jnp.abs(v)
  t = a * rv
  cmin = jnp.minimum(t, jnp.float32(1.0))
  d = cmin * jnp.float32(255.0)
  kp = d.astype(jnp.int32) + lane_base_r
  tb = plsc.load_gather(trep, [kp])
  pa = lax.bitcast_convert_type(a, jnp.int32)
  return kp - (pa < tb).astype(jnp.int32)


def _make_hist256():
  hsz = LANES * LSTRIDE + LANES

  @functools.partial(
      pl.kernel,
      out_type=jax.ShapeDtypeStruct((NW * C * 256,), jnp.int32),
      mesh=_MESH,
      compiler_params=_PARAMS,
      scratch_types=[
          pltpu.VMEM((2, 1, CHUNK), jnp.float32),
          pltpu.VMEM((hsz,), jnp.int32),
          pltpu.VMEM((hsz,), jnp.int32),
          pltpu.VMEM((C * 256,), jnp.int32),
          pltpu.VMEM((LANES * LSTRIDE,), jnp.int32),
          pltpu.VMEM((C * LANES,), jnp.float32),
          pltpu.VMEM((256,), jnp.int32),
          pltpu.SemaphoreType.DMA,
          pltpu.SemaphoreType.DMA,
      ],
  )
  def kfn(x_hbm, tbits_hbm, rbias_hbm, out_hbm, buf, hist0, hist1, tstage,
          trep, rbias_v, merged, sem0, sem1):
    wid = lax.axis_index("c") * NS + lax.axis_index("s")
    pltpu.sync_copy(tbits_hbm, tstage)
    pltpu.sync_copy(rbias_hbm, rbias_v)
    _replicate_table(tstage, trep)
    lane_base = lax.iota(jnp.int32, LANES) * LSTRIDE
    ones = jnp.ones((LANES,), jnp.int32)
    zer = jnp.zeros((LANES,), jnp.int32)

    @plsc.parallel_loop(0, hsz // LANES, unroll=8)
    def _(j):
      hist0[pl.ds(j * LANES, LANES)] = zer
      hist1[pl.ds(j * LANES, LANES)] = zer

    row_state = {}

    def pro(r):
      row_state["rv"] = rbias_v[pl.ds(r * LANES, LANES)]
      row_state["base"] = lane_base + (r * TSTRIDE)

    def one(hist, b, i):
      v = buf[b, 0, pl.ds(i * LANES, LANES)]
      k = _exact_bin(v, row_state["rv"], row_state["base"], trep)
      plsc.addupdate_scatter(hist, [k], ones)

    def body2(r, b, i):
      one(hist0, b, i)
      one(hist1, b, i + 1)

    def epi(r):
      @plsc.parallel_loop(0, 256 // LANES)
      def _(jb):
        o = r * TSTRIDE + jb * LANES
        acc = hist0[pl.ds(o, LANES)] + hist1[pl.ds(o, LANES)]
        for l in range(1, LANES):
          acc = acc + hist0[pl.ds(l * LSTRIDE + o, LANES)]
          acc = acc + hist1[pl.ds(l * LSTRIDE + o, LANES)]
        merged[pl.ds(jb * LANES, LANES)] = acc
      pltpu.sync_copy(merged, out_hbm.at[pl.ds((wid * C + r) * 256, 256)])

    _stream_rows(x_hbm, buf, (sem0, sem1), wid, body2, pro, epi)

  return kfn


def _make_apply():
  @functools.partial(
      pl.kernel,
      out_type=jax.ShapeDtypeStruct((8, C, LROW // 8), jnp.float32),
      mesh=_MESH,
      compiler_params=_PARAMS,
      scratch_types=[
          pltpu.VMEM((2, 1, CHUNK), jnp.float32),
          pltpu.VMEM((2, 1, CHUNK), jnp.float32),
          pltpu.VMEM((C * 256,), jnp.int32),
          pltpu.VMEM((LANES * LSTRIDE,), jnp.int32),
          pltpu.VMEM((C * 256,), jnp.float32),
          pltpu.VMEM((LANES * LSTRIDE,), jnp.float32),
          pltpu.VMEM((C * LANES,), jnp.float32),
          pltpu.SemaphoreType.DMA,
          pltpu.SemaphoreType.DMA,
          pltpu.SemaphoreType.DMA,
          pltpu.SemaphoreType.DMA,
      ],
  )
  def kfn(x_hbm, tbits_hbm, mtab_hbm, rbias_hbm, out_hbm, buf, obuf, tstage,
          trep, mstage, mrep, rbias_v, si0, si1, so0, so1):
    wid = lax.axis_index("c") * NS + lax.axis_index("s")
    pltpu.sync_copy(tbits_hbm, tstage)
    pltpu.sync_copy(mtab_hbm, mstage)
    pltpu.sync_copy(rbias_hbm, rbias_v)
    _replicate_table(tstage, trep)
    _replicate_table(mstage, mrep)
    lane_base = lax.iota(jnp.int32, LANES) * LSTRIDE
    isems = (si0, si1)
    osems = (so0, so1)

    for r in range(C):
      rv = rbias_v[pl.ds(r * LANES, LANES)]
      lane_base_r = lane_base + (r * TSTRIDE)
      pltpu.async_copy(_chunk_slice(x_hbm, wid, r, 0), buf.at[0], isems[0])
      pltpu.async_copy(_chunk_slice(x_hbm, wid, r, 1), buf.at[1], isems[1])

      @pl.loop(0, NCHUNK, step=2)
      def _(g2):
        for b in range(2):
          g = g2 + b
          pltpu.make_async_copy(
              x_hbm.at[0, pl.ds(0, 1), pl.ds(0, CHUNK)], buf.at[b], isems[b]).wait()

          @pl.when(g >= 2)
          def _():
            pltpu.make_async_copy(
                obuf.at[b], out_hbm.at[0, pl.ds(0, 1), pl.ds(0, CHUNK)],
                osems[b]).wait()

          @plsc.parallel_loop(0, CHUNK // LANES, unroll=8)
          def _(i):
            v = buf[b, 0, pl.ds(i * LANES, LANES)]
            k = _exact_bin(v, rv, lane_base_r, trep)
            gt = plsc.load_gather(mrep, [k])
            obuf[b, 0, pl.ds(i * LANES, LANES)] = v * gt

          pltpu.async_copy(
              obuf.at[b], _chunk_slice(out_hbm, wid, r, g), osems[b])
          nxt = g + 2

          @pl.when(nxt < NCHUNK)
          def _():
            pltpu.async_copy(
                _chunk_slice(x_hbm, wid, r, nxt), buf.at[b], isems[b])

      for b in range(2):
        pltpu.make_async_copy(
            obuf.at[b], out_hbm.at[0, pl.ds(0, 1), pl.ds(0, CHUNK)], osems[b]).wait()

  return kfn


_L1 = _make_refine(20, 11, masked=False)
_L2 = _make_refine(9, 11, masked=True)
_L3 = _make_refine(0, 9, masked=True)
_H256 = _make_hist256()
_APPLY = _make_apply()


def _pick(cnt, rank):
  cum = jnp.cumsum(cnt, axis=1)
  b = jnp.argmax(cum >= (rank[:, None] + 1), axis=1).astype(jnp.int32)
  cumprev = jnp.take_along_axis(cum - cnt, b[:, None], axis=1)[:, 0]
  return b, rank - cumprev


def _ref_bin_of_bits(pb, maxv):
  v = lax.bitcast_convert_type(pb, jnp.float32)
  t = v / maxv
  n = t * jnp.float32(8.0)
  cc = jnp.clip(n, jnp.float32(0.0), jnp.float32(8.0))
  ix = ((cc / jnp.float32(8.0)) * jnp.float32(255.0)).astype(jnp.int32)
  return jnp.clip(ix, 0, 255)


def _bin_thresholds(maxv):
  ks = jnp.arange(256, dtype=jnp.int32)[None, :]
  lo = jnp.zeros((C, 256), jnp.int32)
  hi = jnp.full((C, 256), 0x7F800000, jnp.int32)
  mv = maxv[:, None]
  for _ in range(31):
    mid = lo + ((hi - lo) >> 1)
    ok = _ref_bin_of_bits(mid, mv) >= ks
    hi = jnp.where(ok, mid, hi)
    lo = jnp.where(ok, lo, mid + 1)
  return hi.at[:, 0].set(0)


def kernel(x, hist, logp_ref):
  xf = x

  cnt1 = jnp.sum(jnp.reshape(_L1(xf), (NW, C, 2048)), axis=0)
  b1, r1 = _pick(cnt1, jnp.full((C,), RANK, jnp.int32))

  prev2 = jnp.reshape(jnp.broadcast_to(b1[:, None], (C, LANES)), (-1,))
  cnt2 = jnp.sum(jnp.reshape(_L2(xf, prev2), (NW, C, 2048)), axis=0)
  b2, r2 = _pick(cnt2, r1)

  prev3 = jnp.reshape(
      jnp.broadcast_to(((b1 << 11) | b2)[:, None], (C, LANES)), (-1,))
  cnt3 = jnp.sum(jnp.reshape(_L3(xf, prev3), (NW, C, 512)), axis=0)
  b3, _ = _pick(cnt3, r2)

  bits = (b1 << 20) | (b2 << 9) | b3
  maxv = jnp.maximum(
      lax.bitcast_convert_type(bits.astype(jnp.int32), jnp.float32),
      jnp.float32(1e-8))

  tbits = jnp.reshape(_bin_thresholds(maxv), (-1,))
  rbias = (jnp.float32(1.0) / maxv) * jnp.float32(1.0 + 2.0**-20)
  rbias_b = jnp.reshape(jnp.broadcast_to(rbias[:, None], (C, LANES)), (-1,))

  counts = jnp.sum(
      jnp.reshape(_H256(xf, tbits, rbias_b), (NW, C, 256)),
      axis=0).astype(jnp.float32)

  hist2 = (1.0 - 0.02) * hist + 0.02 * counts
  sm = hist2 + 1e-8
  logp_obs = jnp.log(sm / jnp.sum(sm, axis=-1, keepdims=True))
  mask_tab = jax.nn.sigmoid(-1.0 * ((logp_ref - logp_obs) - (-2.0)))

  return _APPLY(xf, tbits, jnp.reshape(mask_tab, (-1,)), rbias_b)


# speedup vs baseline: 5.2933x; 1.0413x over previous
import functools

import jax
import jax.numpy as jnp
from jax import lax
from jax.experimental import pallas as pl
from jax.experimental.pallas import tpu as pltpu
from jax.experimental.pallas import tpu_sc as plsc

NC, NS, LANES = 2, 16, 16
NW = NC * NS
C = 5
LROW = 4194304
NTOT = C * LROW
PER_TILE = LROW // NW
CHUNK = 16384
NCHUNK = PER_TILE // CHUNK
RANK = 4152360
TSTRIDE = 257
LSTRIDE = C * TSTRIDE

_MESH = plsc.VectorSubcoreMesh(
    core_axis_name="c", subcore_axis_name="s", num_cores=NC, num_subcores=NS)
_PARAMS = pltpu.CompilerParams(needs_layout_passes=False)


def _chunk_slice(x_hbm, wid, r, g):
  q = (r * 8) + g
  b = (q * 13) >> 6
  c = q - b * 5
  off = wid * CHUNK
  return x_hbm.at[b, pl.ds(c, 1), pl.ds(off, CHUNK)]


def _stream_rows(x_hbm, buf, sems, wid, body2, row_prologue, row_epilogue):
  for r in range(C):
    row_prologue(r)
    pltpu.async_copy(_chunk_slice(x_hbm, wid, r, 0), buf.at[0], sems[0])
    pltpu.async_copy(_chunk_slice(x_hbm, wid, r, 1), buf.at[1], sems[1])

    @pl.loop(0, NCHUNK, step=2)
    def _(g2):
      for b in range(2):
        g = g2 + b
        pltpu.make_async_copy(
            x_hbm.at[0, pl.ds(0, 1), pl.ds(0, CHUNK)], buf.at[b], sems[b]).wait()

        @plsc.parallel_loop(0, CHUNK // LANES, step=2, unroll=4)
        def _(i):
          body2(r, b, i)

        nxt = g + 2

        @pl.when(nxt < NCHUNK)
        def _():
          pltpu.async_copy(
              _chunk_slice(x_hbm, wid, r, nxt), buf.at[b], sems[b])

    row_epilogue(r)


def _make_refine(shift, nbits, masked):
  nbins = 1 << nbits
  stride = nbins + 1
  hsz = LANES * stride + LANES

  scratch = [
      pltpu.VMEM((2, 1, CHUNK), jnp.float32),
      pltpu.VMEM((hsz,), jnp.int32),
      pltpu.VMEM((hsz,), jnp.int32),
      pltpu.VMEM((nbins,), jnp.int32),
  ]
  if masked:
    scratch.append(pltpu.VMEM((C * LANES,), jnp.int32))
  scratch += [pltpu.SemaphoreType.DMA, pltpu.SemaphoreType.DMA]

  @functools.partial(
      pl.kernel,
      out_type=jax.ShapeDtypeStruct((NW * C * nbins,), jnp.int32),
      mesh=_MESH,
      compiler_params=_PARAMS,
      scratch_types=scratch,
  )
  def kfn(*args):
    if masked:
      (x_hbm, prevp_hbm, out_hbm, buf, hist0, hist1, merged, prevp_v,
       sem0, sem1) = args
    else:
      x_hbm, out_hbm, buf, hist0, hist1, merged, sem0, sem1 = args
      prevp_v = None
    wid = lax.axis_index("c") * NS + lax.axis_index("s")
    if masked:
      pltpu.sync_copy(prevp_hbm, prevp_v)
    lane_base = lax.iota(jnp.int32, LANES) * stride
    ones = jnp.ones((LANES,), jnp.int32)
    zer = jnp.zeros((LANES,), jnp.int32)
    row_state = {}

    def pro(r):
      @plsc.parallel_loop(0, hsz // LANES, unroll=8)
      def _(j):
        hist0[pl.ds(j * LANES, LANES)] = zer
        hist1[pl.ds(j * LANES, LANES)] = zer
      if masked:
        row_state["prev"] = prevp_v[pl.ds(r * LANES, LANES)]

    def one(hist, b, i):
      v = buf[b, 0, pl.ds(i * LANES, LANES)]
      p = lax.bitcast_convert_type(v, jnp.int32) & 0x7FFFFFFF
      digit = ((p >> shift) & (nbins - 1)) + lane_base
      if masked:
        m = (p >> (shift + nbits)) == row_state["prev"]
        plsc.addupdate_scatter(hist, [digit], ones, mask=m)
      else:
        plsc.addupdate_scatter(hist, [digit], ones)

    def body2(r, b, i):
      one(hist0, b, i)
      one(hist1, b, i + 1)

    def epi(r):
      @plsc.parallel_loop(0, nbins // LANES, unroll=2)
      def _(jb):
        o = jb * LANES
        acc = hist0[pl.ds(o, LANES)] + hist1[pl.ds(o, LANES)]
        for l in range(1, LANES):
          acc = acc + hist0[pl.ds(l * stride + o, LANES)]
          acc = acc + hist1[pl.ds(l * stride + o, LANES)]
        merged[pl.ds(o, LANES)] = acc
      pltpu.sync_copy(merged, out_hbm.at[pl.ds((wid * C + r) * nbins, nbins)])

    _stream_rows(x_hbm, buf, (sem0, sem1), wid, body2, pro, epi)

  return kfn


def _replicate_table(src_v, rep_v):
  for r in range(C):
    for jb in range(256 // LANES):
      tv = src_v[pl.ds(r * 256 + jb * LANES, LANES)]
      for l in range(LANES):
        rep_v[pl.ds(l * LSTRIDE + r * TSTRIDE + jb * LANES, LANES)] = tv


def _exact_bin(v, rv, lane_base_r, trep):
  a = jnp.abs(v)
  t = a * rv
  cmin = jnp.minimum(t, jnp.float32(1.0))
  d = cmin * jnp.float32(255.0)
  kp = d.astype(jnp.int32) + lane_base_r
  tb = plsc.load_gather(trep, [kp])
  pa = lax.bitcast_convert_type(a, jnp.int32)
  return kp - (pa < tb).astype(jnp.int32)


def _make_hist256():
  hsz = LANES * LSTRIDE + LANES

  @functools.partial(
      pl.kernel,
      out_type=jax.ShapeDtypeStruct((NW * C * 256,), jnp.int32),
      mesh=_MESH,
      compiler_params=_PARAMS,
      scratch_types=[
          pltpu.VMEM((2, 1, CHUNK), jnp.float32),
          pltpu.VMEM((hsz,), jnp.int32),
          pltpu.VMEM((hsz,), jnp.int32),
          pltpu.VMEM((C * 256,), jnp.int32),
          pltpu.VMEM((LANES * LSTRIDE,), jnp.int32),
          pltpu.VMEM((C * LANES,), jnp.float32),
          pltpu.VMEM((256,), jnp.int32),
          pltpu.SemaphoreType.DMA,
          pltpu.SemaphoreType.DMA,
      ],
  )
  def kfn(x_hbm, tbits_hbm, rbias_hbm, out_hbm, buf, hist0, hist1, tstage,
          trep, rbias_v, merged, sem0, sem1):
    wid = lax.axis_index("c") * NS + lax.axis_index("s")
    pltpu.sync_copy(tbits_hbm, tstage)
    pltpu.sync_copy(rbias_hbm, rbias_v)
    _replicate_table(tstage, trep)
    lane_base = lax.iota(jnp.int32, LANES) * LSTRIDE
    ones = jnp.ones((LANES,), jnp.int32)
    zer = jnp.zeros((LANES,), jnp.int32)

    @plsc.parallel_loop(0, hsz // LANES, unroll=8)
    def _(j):
      hist0[pl.ds(j * LANES, LANES)] = zer
      hist1[pl.ds(j * LANES, LANES)] = zer

    row_state = {}

    def pro(r):
      row_state["rv"] = rbias_v[pl.ds(r * LANES, LANES)]
      row_state["base"] = lane_base + (r * TSTRIDE)

    def one(hist, b, i):
      v = buf[b, 0, pl.ds(i * LANES, LANES)]
      k = _exact_bin(v, row_state["rv"], row_state["base"], trep)
      plsc.addupdate_scatter(hist, [k], ones)

    def body2(r, b, i):
      one(hist0, b, i)
      one(hist1, b, i + 1)

    def epi(r):
      @plsc.parallel_loop(0, 256 // LANES)
      def _(jb):
        o = r * TSTRIDE + jb * LANES
        acc = hist0[pl.ds(o, LANES)] + hist1[pl.ds(o, LANES)]
        for l in range(1, LANES):
          acc = acc + hist0[pl.ds(l * LSTRIDE + o, LANES)]
          acc = acc + hist1[pl.ds(l * LSTRIDE + o, LANES)]
        merged[pl.ds(jb * LANES, LANES)] = acc
      pltpu.sync_copy(merged, out_hbm.at[pl.ds((wid * C + r) * 256, 256)])

    _stream_rows(x_hbm, buf, (sem0, sem1), wid, body2, pro, epi)

  return kfn


def _make_apply():
  @functools.partial(
      pl.kernel,
      out_type=jax.ShapeDtypeStruct((8, C, LROW // 8), jnp.float32),
      mesh=_MESH,
      compiler_params=_PARAMS,
      scratch_types=[
          pltpu.VMEM((2, 1, CHUNK), jnp.float32),
          pltpu.VMEM((2, 1, CHUNK), jnp.float32),
          pltpu.VMEM((C * 256,), jnp.int32),
          pltpu.VMEM((LANES * LSTRIDE,), jnp.int32),
          pltpu.VMEM((C * 256,), jnp.float32),
          pltpu.VMEM((LANES * LSTRIDE,), jnp.float32),
          pltpu.VMEM((C * LANES,), jnp.float32),
          pltpu.SemaphoreType.DMA,
          pltpu.SemaphoreType.DMA,
          pltpu.SemaphoreType.DMA,
          pltpu.SemaphoreType.DMA,
      ],
  )
  def kfn(x_hbm, tbits_hbm, mtab_hbm, rbias_hbm, out_hbm, buf, obuf, tstage,
          trep, mstage, mrep, rbias_v, si0, si1, so0, so1):
    wid = lax.axis_index("c") * NS + lax.axis_index("s")
    pltpu.sync_copy(tbits_hbm, tstage)
    pltpu.sync_copy(mtab_hbm, mstage)
    pltpu.sync_copy(rbias_hbm, rbias_v)
    _replicate_table(tstage, trep)
    _replicate_table(mstage, mrep)
    lane_base = lax.iota(jnp.int32, LANES) * LSTRIDE
    isems = (si0, si1)
    osems = (so0, so1)

    for r in range(C):
      rv = rbias_v[pl.ds(r * LANES, LANES)]
      lane_base_r = lane_base + (r * TSTRIDE)
      pltpu.async_copy(_chunk_slice(x_hbm, wid, r, 0), buf.at[0], isems[0])
      pltpu.async_copy(_chunk_slice(x_hbm, wid, r, 1), buf.at[1], isems[1])

      @pl.loop(0, NCHUNK, step=2)
      def _(g2):
        for b in range(2):
          g = g2 + b
          pltpu.make_async_copy(
              x_hbm.at[0, pl.ds(0, 1), pl.ds(0, CHUNK)], buf.at[b], isems[b]).wait()

          @pl.when(g >= 2)
          def _():
            pltpu.make_async_copy(
                obuf.at[b], out_hbm.at[0, pl.ds(0, 1), pl.ds(0, CHUNK)],
                osems[b]).wait()

          @plsc.parallel_loop(0, CHUNK // LANES, unroll=8)
          def _(i):
            v = buf[b, 0, pl.ds(i * LANES, LANES)]
            k = _exact_bin(v, rv, lane_base_r, trep)
            gt = plsc.load_gather(mrep, [k])
            obuf[b, 0, pl.ds(i * LANES, LANES)] = v * gt

          pltpu.async_copy(
              obuf.at[b], _chunk_slice(out_hbm, wid, r, g), osems[b])
          nxt = g + 2

          @pl.when(nxt < NCHUNK)
          def _():
            pltpu.async_copy(
                _chunk_slice(x_hbm, wid, r, nxt), buf.at[b], isems[b])

      for b in range(2):
        pltpu.make_async_copy(
            obuf.at[b], out_hbm.at[0, pl.ds(0, 1), pl.ds(0, CHUNK)], osems[b]).wait()

  return kfn


_L1 = _make_refine(20, 11, masked=False)
_L2 = _make_refine(9, 11, masked=True)
_L3 = _make_refine(0, 9, masked=True)
_H256 = _make_hist256()
_APPLY = _make_apply()


def _pick(cnt, rank):
  cum = jnp.cumsum(cnt, axis=1)
  b = jnp.argmax(cum >= (rank[:, None] + 1), axis=1).astype(jnp.int32)
  cumprev = jnp.take_along_axis(cum - cnt, b[:, None], axis=1)[:, 0]
  return b, rank - cumprev


def _ref_bin_of_bits(pb, maxv):
  v = lax.bitcast_convert_type(pb, jnp.float32)
  t = v / maxv
  n = t * jnp.float32(8.0)
  cc = jnp.clip(n, jnp.float32(0.0), jnp.float32(8.0))
  ix = ((cc / jnp.float32(8.0)) * jnp.float32(255.0)).astype(jnp.int32)
  return jnp.clip(ix, 0, 255)


def _bin_thresholds(maxv):
  ks = jnp.arange(256, dtype=jnp.int32)[None, :]
  lo = jnp.zeros((C, 256), jnp.int32)
  hi = jnp.full((C, 256), 0x7F800000, jnp.int32)
  mv = maxv[:, None]
  for _ in range(31):
    mid = lo + ((hi - lo) >> 1)
    ok = _ref_bin_of_bits(mid, mv) >= ks
    hi = jnp.where(ok, mid, hi)
    lo = jnp.where(ok, lo, mid + 1)
  return hi.at[:, 0].set(0)


def kernel(x, hist, logp_ref):
  xf = x

  cnt1 = jnp.sum(jnp.reshape(_L1(xf), (NW, C, 2048)), axis=0)
  b1, r1 = _pick(cnt1, jnp.full((C,), RANK, jnp.int32))

  prev2 = jnp.reshape(jnp.broadcast_to(b1[:, None], (C, LANES)), (-1,))
  cnt2 = jnp.sum(jnp.reshape(_L2(xf, prev2), (NW, C, 2048)), axis=0)
  b2, r2 = _pick(cnt2, r1)

  prev3 = jnp.reshape(
      jnp.broadcast_to(((b1 << 11) | b2)[:, None], (C, LANES)), (-1,))
  cnt3 = jnp.sum(jnp.reshape(_L3(xf, prev3), (NW, C, 512)), axis=0)
  b3, _ = _pick(cnt3, r2)

  bits = (b1 << 20) | (b2 << 9) | b3
  maxv = jnp.maximum(
      lax.bitcast_convert_type(bits.astype(jnp.int32), jnp.float32),
      jnp.float32(1e-8))

  tbits = jnp.reshape(_bin_thresholds(maxv), (-1,))
  rbias = (jnp.float32(1.0) / maxv) * jnp.float32(1.0 + 2.0**-20)
  rbias_b = jnp.reshape(jnp.broadcast_to(rbias[:, None], (C, LANES)), (-1,))

  counts = jnp.sum(
      jnp.reshape(_H256(xf, tbits, rbias_b), (NW, C, 256)),
      axis=0).astype(jnp.float32)

  hist2 = (1.0 - 0.02) * hist + 0.02 * counts
  sm = hist2 + 1e-8
  logp_obs = jnp.log(sm / jnp.sum(sm, axis=-1, keepdims=True))
  mask_tab = jax.nn.sigmoid(-1.0 * ((logp_ref - logp_obs) - (-2.0)))

  return _APPLY(xf, tbits, jnp.reshape(mask_tab, (-1,)), rbias_b)
